# baseline (speedup 1.0000x reference)
import functools

import jax
import jax.numpy as jnp
from jax import lax
from jax.experimental import pallas as pl
from jax.experimental.pallas import tpu as pltpu
from jax.experimental.pallas import tpu_sc as plsc

N = 10000
E = 320000
D = 128
NPAD = 10240

NC = 2
NS = 16
NW = NC * NS
EPT = E // NW
CHUNK = 40
NCHUNKS = EPT // CHUNK
RPS = NPAD // NS

_mesh = plsc.VectorSubcoreMesh(core_axis_name="c", subcore_axis_name="s")


@functools.partial(
    pl.kernel,
    out_type=jax.ShapeDtypeStruct((NW, NPAD), jnp.float32),
    mesh=_mesh,
    compiler_params=pltpu.CompilerParams(needs_layout_passes=False),
    scratch_types=[
        pltpu.VMEM((EPT,), jnp.int32),
        pltpu.VMEM((NPAD,), jnp.float32),
    ],
)
def _deg(src_hbm, deg_out, sidx, hist):
    cid = lax.axis_index("c")
    sid = lax.axis_index("s")
    wid = cid * NS + sid

    def zbody(i, carry):
        hist[pl.ds(i * 16, 16)] = jnp.zeros((16,), jnp.float32)
        return carry

    lax.fori_loop(0, NPAD // 16, zbody, 0)
    pltpu.sync_copy(src_hbm.at[pl.ds(wid * EPT, EPT)], sidx)

    ones16 = jnp.ones((16,), jnp.float32)

    def body(i, carry):
        idx16 = sidx[pl.ds(i * 16, 16)]
        plsc.addupdate_scatter(hist, [idx16], ones16)
        return carry

    lax.fori_loop(0, EPT // 16, body, 0)
    pltpu.sync_copy(hist, deg_out.at[wid])


def _matmul_body(x_ref, w_ref, xw_ref):
    xw_ref[...] = jnp.dot(x_ref[...], w_ref[...],
                          preferred_element_type=jnp.float32)


def _scale_body(xw_ref, degp_ref, y_ref):
    deg = jnp.sum(degp_ref[...], axis=0)
    scale = jnp.where(deg > 0, 1.0 / deg, 0.0)
    y_ref[pl.ds(0, N), :] = xw_ref[...] * scale[:N, None]
    y_ref[pl.ds(N, NPAD - N), :] = jnp.zeros((NPAD - N, D), jnp.float32)


NBUF = 5
LOOKAHEAD = 2
NGRP = NCHUNKS // NBUF


@functools.partial(
    pl.kernel,
    out_type=jax.ShapeDtypeStruct((NC, NPAD, D), jnp.float32),
    mesh=_mesh,
    scratch_types=(
        [pltpu.VMEM((EPT,), jnp.int32)]
        + [pltpu.VMEM((CHUNK,), jnp.int32) for _ in range(NBUF)]
        + [pltpu.VMEM((CHUNK, D), jnp.float32) for _ in range(NBUF)]
        + [pltpu.VMEM_SHARED((NPAD, D), jnp.float32)]
        + [pltpu.SemaphoreType.DMA for _ in range(3 * NBUF)]
    ),
)
def _agg(y_hbm, src_hbm, dst_hbm, zeros_hbm, out_hbm, *sc):
    sidx = sc[0]
    didx = sc[1:1 + NBUF]
    rows = sc[1 + NBUF:1 + 2 * NBUF]
    acc = sc[1 + 2 * NBUF]
    gsem = sc[2 + 2 * NBUF:2 + 3 * NBUF]
    dsem = sc[2 + 3 * NBUF:2 + 4 * NBUF]
    ssem = sc[2 + 4 * NBUF:2 + 5 * NBUF]

    cid = lax.axis_index("c")
    sid = lax.axis_index("s")
    wid = cid * NS + sid
    ebase = wid * EPT

    pltpu.sync_copy(src_hbm.at[pl.ds(ebase, EPT)], sidx)
    pltpu.sync_copy(zeros_hbm.at[pl.ds(sid * RPS, RPS), :],
                    acc.at[pl.ds(sid * RPS, RPS), :])
    plsc.subcore_barrier()

    def gather_start(j, b):
        pltpu.async_copy(dst_hbm.at[pl.ds(ebase + j * CHUNK, CHUNK)],
                         didx[b], dsem[b])
        pltpu.async_copy(y_hbm.at[sidx.at[pl.ds(j * CHUNK, CHUNK)]],
                         rows[b], gsem[b])

    def gather_wait(j, b):
        pltpu.make_async_copy(dst_hbm.at[pl.ds(ebase + j * CHUNK, CHUNK)],
                              didx[b], dsem[b]).wait()
        pltpu.make_async_copy(y_hbm.at[sidx.at[pl.ds(j * CHUNK, CHUNK)]],
                              rows[b], gsem[b]).wait()

    def scatter_wait(b):
        pltpu.make_async_copy(rows[b], acc.at[didx[b]], ssem[b]).wait()

    for b in range(LOOKAHEAD):
        gather_start(b, b)

    def body(o, carry):
        for b in range(NBUF):
            j = o * NBUF + b
            gather_wait(j, b)
            pltpu.async_copy(rows[b], acc.at[didx[b]], ssem[b], add=True)
            b2 = (b + LOOKAHEAD) % NBUF

            @pl.when(j + LOOKAHEAD < NCHUNKS)
            def _():
                @pl.when(j + LOOKAHEAD >= NBUF)
                def _():
                    scatter_wait(b2)

                gather_start(j + LOOKAHEAD, b2)

        return carry

    lax.fori_loop(0, NGRP, body, 0)
    for j in range(NCHUNKS - NBUF, NCHUNKS):
        scatter_wait(j % NBUF)
    plsc.subcore_barrier()
    pltpu.sync_copy(acc.at[pl.ds(sid * RPS, RPS), :],
                    out_hbm.at[cid, pl.ds(sid * RPS, RPS), :])


def _combine_body(p_ref, b_ref, out_ref):
    out_ref[...] = p_ref[0, :N, :] + p_ref[1, :N, :] + b_ref[...]


def kernel(x, edge_index, W, b):
    src = edge_index[0]
    dst = edge_index[1]

    zeros_mat = jnp.zeros((NPAD, D), jnp.float32)

    degp = _deg(src)

    xw = pl.pallas_call(
        _matmul_body,
        out_shape=jax.ShapeDtypeStruct((N, D), jnp.float32),
    )(x, W)

    y = pl.pallas_call(
        _scale_body,
        out_shape=jax.ShapeDtypeStruct((NPAD, D), jnp.float32),
    )(xw, degp)

    partials = _agg(y, src, dst, zeros_mat)

    out = pl.pallas_call(
        _combine_body,
        out_shape=jax.ShapeDtypeStruct((N, D), jnp.float32),
    )(partials, b)
    return out


# speedup vs baseline: 1.2820x; 1.2820x over previous
import functools

import jax
import jax.numpy as jnp
from jax import lax
from jax.experimental import pallas as pl
from jax.experimental.pallas import tpu as pltpu
from jax.experimental.pallas import tpu_sc as plsc

N = 10000
E = 320000
D = 128
NPAD = 10240

NC = 2
NS = 16
NW = NC * NS
EPT = E // NW
CHUNK = 80
NCHUNKS = EPT // CHUNK
RPS = NPAD // NS

_mesh = plsc.VectorSubcoreMesh(core_axis_name="c", subcore_axis_name="s")


@functools.partial(
    pl.kernel,
    out_type=jax.ShapeDtypeStruct((NW, NPAD), jnp.float32),
    mesh=_mesh,
    compiler_params=pltpu.CompilerParams(needs_layout_passes=False),
    scratch_types=[
        pltpu.VMEM((EPT,), jnp.int32),
        pltpu.VMEM((NPAD,), jnp.float32),
    ],
)
def _deg(src_hbm, deg_out, sidx, hist):
    cid = lax.axis_index("c")
    sid = lax.axis_index("s")
    wid = cid * NS + sid

    def zbody(i, carry):
        hist[pl.ds(i * 16, 16)] = jnp.zeros((16,), jnp.float32)
        return carry

    lax.fori_loop(0, NPAD // 16, zbody, 0)
    pltpu.sync_copy(src_hbm.at[pl.ds(wid * EPT, EPT)], sidx)

    ones16 = jnp.ones((16,), jnp.float32)

    def body(i, carry):
        idx16 = sidx[pl.ds(i * 16, 16)]
        plsc.addupdate_scatter(hist, [idx16], ones16)
        return carry

    lax.fori_loop(0, EPT // 16, body, 0)
    pltpu.sync_copy(hist, deg_out.at[wid])


def _matmul_body(x_ref, w_ref, xw_ref):
    xw_ref[...] = jnp.dot(x_ref[...], w_ref[...],
                          preferred_element_type=jnp.float32)


def _scale_body(xw_ref, degp_ref, y_ref):
    deg = jnp.sum(degp_ref[...], axis=0)
    scale = jnp.where(deg > 0, 1.0 / deg, 0.0)
    y_ref[pl.ds(0, N), :] = xw_ref[...] * scale[:N, None]
    y_ref[pl.ds(N, NPAD - N), :] = jnp.zeros((NPAD - N, D), jnp.float32)


NBUF = 3
NGRP = NCHUNKS // NBUF
NTAIL = NCHUNKS - NGRP * NBUF


@functools.partial(
    pl.kernel,
    out_type=jax.ShapeDtypeStruct((NC, NPAD, D), jnp.float32),
    mesh=_mesh,
    scratch_types=(
        [pltpu.VMEM((EPT,), jnp.int32)]
        + [pltpu.VMEM((CHUNK,), jnp.int32) for _ in range(NBUF)]
        + [pltpu.VMEM((CHUNK, D), jnp.float32) for _ in range(NBUF)]
        + [pltpu.VMEM_SHARED((NPAD, D), jnp.float32)]
        + [pltpu.SemaphoreType.DMA for _ in range(2 * NBUF)]
    ),
)
def _agg(y_hbm, src_hbm, dst_hbm, zeros_hbm, out_hbm, *sc):
    sidx = sc[0]
    didx = sc[1:1 + NBUF]
    rows = sc[1 + NBUF:1 + 2 * NBUF]
    acc = sc[1 + 2 * NBUF]
    gsem = sc[2 + 2 * NBUF:2 + 3 * NBUF]
    dsem = sc[2 + 3 * NBUF:2 + 4 * NBUF]

    cid = lax.axis_index("c")
    sid = lax.axis_index("s")
    wid = cid * NS + sid
    ebase = wid * EPT

    pltpu.sync_copy(src_hbm.at[pl.ds(ebase, EPT)], sidx)
    pltpu.sync_copy(zeros_hbm.at[pl.ds(sid * RPS, RPS), :],
                    acc.at[pl.ds(sid * RPS, RPS), :])
    plsc.subcore_barrier()

    def gather_start(j, b):
        pltpu.async_copy(dst_hbm.at[pl.ds(ebase + j * CHUNK, CHUNK)],
                         didx[b], dsem[b])
        pltpu.async_copy(y_hbm.at[sidx.at[pl.ds(j * CHUNK, CHUNK)]],
                         rows[b], gsem[b])

    def gather_wait(j, b):
        pltpu.make_async_copy(dst_hbm.at[pl.ds(ebase + j * CHUNK, CHUNK)],
                              didx[b], dsem[b]).wait()
        pltpu.make_async_copy(y_hbm.at[sidx.at[pl.ds(j * CHUNK, CHUNK)]],
                              rows[b], gsem[b]).wait()

    for b in range(NBUF):
        gather_start(b, b)

    def body(o, carry):
        for b in range(NBUF):
            j = o * NBUF + b
            gather_wait(j, b)
            pltpu.sync_copy(rows[b], acc.at[didx[b]], add=True)

            @pl.when(j + NBUF < NCHUNKS)
            def _():
                gather_start(j + NBUF, b)

        return carry

    lax.fori_loop(0, NGRP, body, 0)
    for t in range(NTAIL):
        j = NGRP * NBUF + t
        gather_wait(j, t)
        pltpu.sync_copy(rows[t], acc.at[didx[t]], add=True)
    plsc.subcore_barrier()
    pltpu.sync_copy(acc.at[pl.ds(sid * RPS, RPS), :],
                    out_hbm.at[cid, pl.ds(sid * RPS, RPS), :])


def _combine_body(p_ref, b_ref, out_ref):
    out_ref[...] = p_ref[0, :N, :] + p_ref[1, :N, :] + b_ref[...]


def kernel(x, edge_index, W, b):
    src = edge_index[0]
    dst = edge_index[1]

    zeros_mat = jnp.zeros((NPAD, D), jnp.float32)

    degp = _deg(src)

    xw = pl.pallas_call(
        _matmul_body,
        out_shape=jax.ShapeDtypeStruct((N, D), jnp.float32),
    )(x, W)

    y = pl.pallas_call(
        _scale_body,
        out_shape=jax.ShapeDtypeStruct((NPAD, D), jnp.float32),
    )(xw, degp)

    partials = _agg(y, src, dst, zeros_mat)

    out = pl.pallas_call(
        _combine_body,
        out_shape=jax.ShapeDtypeStruct((N, D), jnp.float32),
    )(partials, b)
    return out


# speedup vs baseline: 1.3241x; 1.0328x over previous
import functools

import jax
import jax.numpy as jnp
from jax import lax
from jax.experimental import pallas as pl
from jax.experimental.pallas import tpu as pltpu
from jax.experimental.pallas import tpu_sc as plsc

N = 10000
E = 320000
D = 128
NPAD = 10240

NC = 2
NS = 16
NW = NC * NS
EPT = E // NW
CHUNK = 40
NCHUNKS = EPT // CHUNK
RPS = NPAD // NS

_mesh = plsc.VectorSubcoreMesh(core_axis_name="c", subcore_axis_name="s")


@functools.partial(
    pl.kernel,
    out_type=jax.ShapeDtypeStruct((NW, NPAD), jnp.float32),
    mesh=_mesh,
    compiler_params=pltpu.CompilerParams(needs_layout_passes=False),
    scratch_types=[
        pltpu.VMEM((EPT,), jnp.int32),
        pltpu.VMEM((NPAD,), jnp.float32),
    ],
)
def _deg(src_hbm, deg_out, sidx, hist):
    cid = lax.axis_index("c")
    sid = lax.axis_index("s")
    wid = cid * NS + sid

    def zbody(i, carry):
        hist[pl.ds(i * 16, 16)] = jnp.zeros((16,), jnp.float32)
        return carry

    lax.fori_loop(0, NPAD // 16, zbody, 0)
    pltpu.sync_copy(src_hbm.at[pl.ds(wid * EPT, EPT)], sidx)

    ones16 = jnp.ones((16,), jnp.float32)

    def body(i, carry):
        idx16 = sidx[pl.ds(i * 16, 16)]
        plsc.addupdate_scatter(hist, [idx16], ones16)
        return carry

    lax.fori_loop(0, EPT // 16, body, 0)
    pltpu.sync_copy(hist, deg_out.at[wid])


def _matscale_body(x_ref, w_ref, degp_ref, y_ref):
    deg = jnp.sum(degp_ref[...], axis=0)
    scale = jnp.where(deg > 0, 1.0 / deg, 0.0)
    xw = jnp.dot(x_ref[...], w_ref[...], preferred_element_type=jnp.float32)
    y_ref[pl.ds(0, N), :] = xw * scale[:N, None]
    y_ref[pl.ds(N, NPAD - N), :] = jnp.zeros((NPAD - N, D), jnp.float32)


NBUF = 5
NGRP = NCHUNKS // NBUF
NTAIL = NCHUNKS - NGRP * NBUF


@functools.partial(
    pl.kernel,
    out_type=jax.ShapeDtypeStruct((NC, NPAD, D), jnp.float32),
    mesh=_mesh,
    scratch_types=(
        [pltpu.VMEM((EPT,), jnp.int32)]
        + [pltpu.VMEM((CHUNK,), jnp.int32) for _ in range(NBUF)]
        + [pltpu.VMEM((CHUNK, D), jnp.float32) for _ in range(NBUF)]
        + [pltpu.VMEM_SHARED((NPAD, D), jnp.float32)]
        + [pltpu.SemaphoreType.DMA for _ in range(2 * NBUF)]
    ),
)
def _agg(y_hbm, src_hbm, dst_hbm, zeros_hbm, out_hbm, *sc):
    sidx = sc[0]
    didx = sc[1:1 + NBUF]
    rows = sc[1 + NBUF:1 + 2 * NBUF]
    acc = sc[1 + 2 * NBUF]
    gsem = sc[2 + 2 * NBUF:2 + 3 * NBUF]
    dsem = sc[2 + 3 * NBUF:2 + 4 * NBUF]

    cid = lax.axis_index("c")
    sid = lax.axis_index("s")
    wid = cid * NS + sid
    ebase = wid * EPT

    pltpu.sync_copy(src_hbm.at[pl.ds(ebase, EPT)], sidx)
    pltpu.sync_copy(zeros_hbm.at[pl.ds(sid * RPS, RPS), :],
                    acc.at[pl.ds(sid * RPS, RPS), :])
    plsc.subcore_barrier()

    def gather_start(j, b):
        pltpu.async_copy(dst_hbm.at[pl.ds(ebase + j * CHUNK, CHUNK)],
                         didx[b], dsem[b])
        pltpu.async_copy(y_hbm.at[sidx.at[pl.ds(j * CHUNK, CHUNK)]],
                         rows[b], gsem[b])

    def gather_wait(j, b):
        pltpu.make_async_copy(dst_hbm.at[pl.ds(ebase + j * CHUNK, CHUNK)],
                              didx[b], dsem[b]).wait()
        pltpu.make_async_copy(y_hbm.at[sidx.at[pl.ds(j * CHUNK, CHUNK)]],
                              rows[b], gsem[b]).wait()

    for b in range(NBUF):
        gather_start(b, b)

    def body(o, carry):
        for b in range(NBUF):
            j = o * NBUF + b
            gather_wait(j, b)
            pltpu.sync_copy(rows[b], acc.at[didx[b]], add=True)

            @pl.when(j + NBUF < NCHUNKS)
            def _():
                gather_start(j + NBUF, b)

        return carry

    lax.fori_loop(0, NGRP, body, 0)
    for t in range(NTAIL):
        j = NGRP * NBUF + t
        gather_wait(j, t)
        pltpu.sync_copy(rows[t], acc.at[didx[t]], add=True)
    plsc.subcore_barrier()
    pltpu.sync_copy(acc.at[pl.ds(sid * RPS, RPS), :],
                    out_hbm.at[cid, pl.ds(sid * RPS, RPS), :])


def _combine_body(p_ref, b_ref, out_ref):
    out_ref[...] = p_ref[0, :N, :] + p_ref[1, :N, :] + b_ref[...]


def kernel(x, edge_index, W, b):
    src = edge_index[0]
    dst = edge_index[1]

    zeros_mat = jnp.zeros((NPAD, D), jnp.float32)

    degp = _deg(src)

    y = pl.pallas_call(
        _matscale_body,
        out_shape=jax.ShapeDtypeStruct((NPAD, D), jnp.float32),
    )(x, W, degp)

    partials = _agg(y, src, dst, zeros_mat)

    out = pl.pallas_call(
        _combine_body,
        out_shape=jax.ShapeDtypeStruct((N, D), jnp.float32),
    )(partials, b)
    return out


# speedup vs baseline: 1.3424x; 1.0138x over previous
import functools

import jax
import jax.numpy as jnp
from jax import lax
from jax.experimental import pallas as pl
from jax.experimental.pallas import tpu as pltpu
from jax.experimental.pallas import tpu_sc as plsc

N = 10000
E = 320000
D = 128
NPAD = 10240

NC = 2
NS = 16
NW = NC * NS
EPT = E // NW
CHUNK = 40
NCHUNKS = EPT // CHUNK
RPS = NPAD // NS

_mesh = plsc.VectorSubcoreMesh(core_axis_name="c", subcore_axis_name="s")


@functools.partial(
    pl.kernel,
    out_type=jax.ShapeDtypeStruct((NW, NPAD), jnp.float32),
    mesh=_mesh,
    compiler_params=pltpu.CompilerParams(needs_layout_passes=False),
    scratch_types=[
        pltpu.VMEM((EPT,), jnp.int32),
        pltpu.VMEM((NPAD,), jnp.float32),
        pltpu.SemaphoreType.DMA,
    ],
)
def _deg(src_hbm, deg_out, sidx, hist, hsem):
    cid = lax.axis_index("c")
    sid = lax.axis_index("s")
    wid = cid * NS + sid

    cp = pltpu.async_copy(src_hbm.at[pl.ds(wid * EPT, EPT)], sidx, hsem)

    def zbody(i, carry):
        hist[pl.ds(i * 16, 16)] = jnp.zeros((16,), jnp.float32)
        return carry

    lax.fori_loop(0, NPAD // 16, zbody, 0)
    cp.wait()

    ones16 = jnp.ones((16,), jnp.float32)

    def body(i, carry):
        idx16 = sidx[pl.ds(i * 16, 16)]
        plsc.addupdate_scatter(hist, [idx16], ones16)
        return carry

    lax.fori_loop(0, EPT // 16, body, 0)
    pltpu.sync_copy(hist, deg_out.at[wid])


def _matscale_body(x_ref, w_ref, degp_ref, y_ref):
    deg = jnp.sum(degp_ref[...], axis=0)
    scale = jnp.where(deg > 0, 1.0 / deg, 0.0)
    xw = jnp.dot(x_ref[...], w_ref[...], preferred_element_type=jnp.float32)
    y_ref[pl.ds(0, N), :] = xw * scale[:N, None]
    y_ref[pl.ds(N, NPAD - N), :] = jnp.zeros((NPAD - N, D), jnp.float32)


NBUF = 5
NGRP = NCHUNKS // NBUF
NTAIL = NCHUNKS - NGRP * NBUF


@functools.partial(
    pl.kernel,
    out_type=jax.ShapeDtypeStruct((NC, NPAD, D), jnp.float32),
    mesh=_mesh,
    scratch_types=(
        [pltpu.VMEM((EPT,), jnp.int32)]
        + [pltpu.VMEM((CHUNK,), jnp.int32) for _ in range(NBUF)]
        + [pltpu.VMEM((CHUNK, D), jnp.float32) for _ in range(NBUF)]
        + [pltpu.VMEM_SHARED((NPAD, D), jnp.float32)]
        + [pltpu.SemaphoreType.DMA for _ in range(2 * NBUF + 1)]
    ),
)
def _agg(y_hbm, src_hbm, dst_hbm, zeros_hbm, out_hbm, *sc):
    sidx = sc[0]
    didx = sc[1:1 + NBUF]
    rows = sc[1 + NBUF:1 + 2 * NBUF]
    acc = sc[1 + 2 * NBUF]
    gsem = sc[2 + 2 * NBUF:2 + 3 * NBUF]
    dsem = sc[2 + 3 * NBUF:2 + 4 * NBUF]
    zsem = sc[2 + 4 * NBUF]

    cid = lax.axis_index("c")
    sid = lax.axis_index("s")
    wid = cid * NS + sid
    ebase = wid * EPT

    pltpu.sync_copy(src_hbm.at[pl.ds(ebase, EPT)], sidx)
    zcp = pltpu.async_copy(zeros_hbm.at[pl.ds(sid * RPS, RPS), :],
                           acc.at[pl.ds(sid * RPS, RPS), :], zsem)

    def gather_start(j, b):
        pltpu.async_copy(dst_hbm.at[pl.ds(ebase + j * CHUNK, CHUNK)],
                         didx[b], dsem[b])
        pltpu.async_copy(y_hbm.at[sidx.at[pl.ds(j * CHUNK, CHUNK)]],
                         rows[b], gsem[b])

    def gather_wait(j, b):
        pltpu.make_async_copy(dst_hbm.at[pl.ds(ebase + j * CHUNK, CHUNK)],
                              didx[b], dsem[b]).wait()
        pltpu.make_async_copy(y_hbm.at[sidx.at[pl.ds(j * CHUNK, CHUNK)]],
                              rows[b], gsem[b]).wait()

    for b in range(NBUF):
        gather_start(b, b)
    zcp.wait()
    plsc.subcore_barrier()

    def body(o, carry):
        for b in range(NBUF):
            j = o * NBUF + b
            gather_wait(j, b)
            pltpu.sync_copy(rows[b], acc.at[didx[b]], add=True)

            @pl.when(j + NBUF < NCHUNKS)
            def _():
                gather_start(j + NBUF, b)

        return carry

    lax.fori_loop(0, NGRP, body, 0)
    for t in range(NTAIL):
        j = NGRP * NBUF + t
        gather_wait(j, t)
        pltpu.sync_copy(rows[t], acc.at[didx[t]], add=True)
    plsc.subcore_barrier()
    pltpu.sync_copy(acc.at[pl.ds(sid * RPS, RPS), :],
                    out_hbm.at[cid, pl.ds(sid * RPS, RPS), :])


def _combine_body(p_ref, b_ref, out_ref):
    out_ref[...] = p_ref[0, :N, :] + p_ref[1, :N, :] + b_ref[...]


def kernel(x, edge_index, W, b):
    src = edge_index[0]
    dst = edge_index[1]

    zeros_mat = jnp.zeros((NPAD, D), jnp.float32)

    degp = _deg(src)

    y = pl.pallas_call(
        _matscale_body,
        out_shape=jax.ShapeDtypeStruct((NPAD, D), jnp.float32),
    )(x, W, degp)

    partials = _agg(y, src, dst, zeros_mat)

    out = pl.pallas_call(
        _combine_body,
        out_shape=jax.ShapeDtypeStruct((N, D), jnp.float32),
    )(partials, b)
    return out
